# baseline (speedup 1.0000x reference)
import jax
import jax.numpy as jnp
from jax.experimental import pallas as pl
from jax.experimental.pallas import tpu as pltpu

B, S, DV, DA = 64, 512, 1024, 512
DM = 2 * DA

BB = 8
SC = 128
NCHUNK = S // SC


def _attn_kernel(vec_ref, mat_ref, len_ref, wv_ref, wm_ref, wa_ref,
                 reps_ref, attn_ref):
    f32 = jnp.float32
    t1 = jnp.dot(vec_ref[...], wv_ref[...], preferred_element_type=f32)

    t1big = jnp.broadcast_to(t1[:, None, :], (BB, SC, DA)).reshape(BB * SC, DA)
    wa = wa_ref[...]

    logit_chunks = []
    for c in range(NCHUNK):
        rows = mat_ref[:, c * SC:(c + 1) * SC, :].reshape(BB * SC, DM)
        t2 = jnp.dot(rows, wm_ref[...], preferred_element_type=f32)
        t3 = jnp.maximum(t2 + t1big, 0.0) * wa
        logit_chunks.append(jnp.sum(t3.reshape(BB, SC, DA), axis=-1))
    logits = jnp.concatenate(logit_chunks, axis=1)

    rowmax = jnp.max(logits, axis=-1, keepdims=True)
    unnorm = jnp.exp(logits - rowmax)
    seq = jax.lax.broadcasted_iota(jnp.int32, (BB, S), 1)
    masked = jnp.where(seq < len_ref[...], unnorm, 0.0)
    denom = jnp.sum(masked, axis=-1, keepdims=True)
    attn = masked / denom
    attn_ref[...] = attn

    sub = jax.lax.broadcasted_iota(jnp.int32, (BB, BB * S), 0)
    blk = jax.lax.broadcasted_iota(jnp.int32, (BB, BB * S), 1) // S
    on_band = sub == blk
    rows_full = mat_ref[...].reshape(BB * S, DM)
    band = jnp.where(on_band, jnp.concatenate([attn] * BB, axis=1), 0.0)
    reps_ref[...] = jnp.dot(band, rows_full, preferred_element_type=f32)


def kernel(vector, matrix, input_lengths, W_vec, W_mat, w_attn):
    lengths = input_lengths.astype(jnp.int32).reshape(B, 1)
    wa2 = w_attn.reshape(1, DA)

    grid = (B // BB,)
    reps, attn = pl.pallas_call(
        _attn_kernel,
        out_shape=(
            jax.ShapeDtypeStruct((B, DM), jnp.float32),
            jax.ShapeDtypeStruct((B, S), jnp.float32),
        ),
        grid=grid,
        in_specs=[
            pl.BlockSpec((BB, DV), lambda i: (i, 0)),
            pl.BlockSpec((BB, S, DM), lambda i: (i, 0, 0)),
            pl.BlockSpec((BB, 1), lambda i: (i, 0)),
            pl.BlockSpec((DV, DA), lambda i: (0, 0)),
            pl.BlockSpec((DM, DA), lambda i: (0, 0)),
            pl.BlockSpec((1, DA), lambda i: (0, 0)),
        ],
        out_specs=(
            pl.BlockSpec((BB, DM), lambda i: (i, 0)),
            pl.BlockSpec((BB, S), lambda i: (i, 0)),
        ),
        compiler_params=pltpu.CompilerParams(
            dimension_semantics=("arbitrary",),
            vmem_limit_bytes=50 * 1024 * 1024,
        ),
        name="attention_pair",
    )(vector, matrix, lengths, W_vec, W_mat, wa2)
    return reps, attn
